# baseline (speedup 1.0000x reference)
import functools

import jax
import jax.numpy as jnp
from jax import lax
from jax.experimental import pallas as pl
from jax.experimental.pallas import tpu as pltpu
from jax.experimental.pallas import tpu_sc as plsc

B_TOK = 4096 * 200
EMB_D = 64
CH = 128
K = 8
GROUP = CH * K
NW = 32
BPW = B_TOK // NW
NG = BPW // GROUP

_mesh = plsc.VectorSubcoreMesh(core_axis_name="c", subcore_axis_name="s")


@functools.partial(
    pl.kernel,
    mesh=_mesh,
    out_type=jax.ShapeDtypeStruct((B_TOK, EMB_D), jnp.float32),
    scratch_types=[
        pltpu.VMEM((K, CH), jnp.int32),
        pltpu.VMEM((GROUP, EMB_D), jnp.float32),
        pltpu.SemaphoreType.DMA,
    ],
    compiler_params=pltpu.CompilerParams(use_tc_tiling_on_sc=False),
)
def _embed_sc(idx_hbm, table_hbm, out_hbm, idx_v, rows_v, sem):
    wid = lax.axis_index("s") * 2 + lax.axis_index("c")
    row0 = wid * (BPW // CH)

    def body(g, _):
        grow = row0 + g * K
        gbase = grow * CH
        pltpu.sync_copy(idx_hbm.at[pl.ds(grow, K)], idx_v)
        copies = [
            pltpu.make_async_copy(
                table_hbm.at[idx_v.at[j]],
                rows_v.at[pl.ds(j * CH, CH)],
                sem,
            )
            for j in range(K)
        ]
        for c in copies:
            c.start()
        for c in copies:
            c.wait()
        pltpu.sync_copy(rows_v, out_hbm.at[pl.ds(gbase, GROUP)])
        return 0

    lax.fori_loop(0, NG, body, 0)


def kernel(input_, weight):
    idx = input_.reshape(B_TOK // CH, CH).astype(jnp.int32)
    out = _embed_sc(idx, weight)
    return out.reshape(input_.shape[0], input_.shape[1], EMB_D)


# speedup vs baseline: 1.0166x; 1.0166x over previous
import functools

import jax
import jax.numpy as jnp
from jax import lax
from jax.experimental import pallas as pl
from jax.experimental.pallas import tpu as pltpu
from jax.experimental.pallas import tpu_sc as plsc

B_TOK = 4096 * 200
EMB_D = 64
CH = 128
K = 5
GROUP = CH * K
NW = 32
BPW = B_TOK // NW
NG = BPW // GROUP
IR_PW = BPW // CH

_mesh = plsc.VectorSubcoreMesh(core_axis_name="c", subcore_axis_name="s")


@functools.partial(
    pl.kernel,
    mesh=_mesh,
    out_type=jax.ShapeDtypeStruct((B_TOK, EMB_D), jnp.float32),
    scratch_types=[
        pltpu.VMEM((IR_PW, CH), jnp.int32),
        pltpu.VMEM((2, GROUP, EMB_D), jnp.float32),
        pltpu.SemaphoreType.DMA,
        pltpu.SemaphoreType.DMA,
        pltpu.SemaphoreType.DMA,
    ],
    compiler_params=pltpu.CompilerParams(use_tc_tiling_on_sc=False),
)
def _embed_sc(idx_hbm, table_hbm, out_hbm, idx_v, rows_v, gsem, osem0, osem1):
    wid = lax.axis_index("s") * 2 + lax.axis_index("c")
    row0 = wid * IR_PW
    osems = (osem0, osem1)

    def gather_copies(g, b):
        return [
            pltpu.make_async_copy(
                table_hbm.at[idx_v.at[g * K + j]],
                rows_v.at[b].at[pl.ds(j * CH, CH)],
                gsem,
            )
            for j in range(K)
        ]

    def out_copy(g, b):
        return pltpu.make_async_copy(
            rows_v.at[b],
            out_hbm.at[pl.ds((row0 + g * K) * CH, GROUP)],
            osems[b],
        )

    pltpu.sync_copy(idx_hbm.at[pl.ds(row0, IR_PW)], idx_v)
    for c in gather_copies(0, 0):
        c.start()

    def body(s, _):
        for half in range(2):
            b = half
            ob = 1 - b
            g = 2 * s + half
            for c in gather_copies(g, b):
                c.wait()
            out_copy(g, b).start()

            @pl.when(g >= 1)
            def _():
                out_copy(g - 1, ob).wait()

            @pl.when(g + 1 < NG)
            def _():
                for c in gather_copies(g + 1, ob):
                    c.start()

        return 0

    lax.fori_loop(0, NG // 2, body, 0)
    out_copy(NG - 1, (NG - 1) % 2).wait()


def kernel(input_, weight):
    idx = input_.reshape(B_TOK // CH, CH).astype(jnp.int32)
    out = _embed_sc(idx, weight)
    return out.reshape(input_.shape[0], input_.shape[1], EMB_D)
